# Initial kernel scaffold; baseline (speedup 1.0000x reference)
import jax
import jax.numpy as jnp
from jax.experimental import pallas as pl


def kernel(x, edge_index, batch, W1, b1, g1, be1, W2, b2, g2, be2, fcW, fcb):
    raise NotImplementedError("write your pallas kernel here")


# speedup vs baseline: 12.4623x; 12.4623x over previous
import functools

import jax
import jax.numpy as jnp
from jax import lax
from jax.experimental import pallas as pl
from jax.experimental.pallas import tpu as pltpu
from jax.experimental.pallas import tpu_sc as plsc

N = 10000
E = 320000
H = 128
C = 40
G = 64

NC = 2
NS = 16
NT = NC * NS
EPT = E // NT
CHUNK = 80
NCHUNK = EPT // CHUNK
NPAD = 10240
RPT = NPAD // NS

BK = 1000

_mesh = plsc.VectorSubcoreMesh(core_axis_name="c", subcore_axis_name="s")


@functools.partial(
    pl.kernel,
    mesh=_mesh,
    out_type=jax.ShapeDtypeStruct((NC, NPAD), jnp.float32),
    scratch_types=[
        pltpu.VMEM((CHUNK,), jnp.int32),
        pltpu.VMEM((CHUNK,), jnp.float32),
        pltpu.VMEM_SHARED((NPAD,), jnp.float32),
    ],
)
def _sc_deg(dst_hbm, zer_hbm, out_hbm, idx_v, ones_v, acc_sh):
    c = lax.axis_index("c")
    s = lax.axis_index("s")
    pltpu.sync_copy(zer_hbm, acc_sh.at[pl.ds(s * RPT, RPT)])
    for j in range(CHUNK // 16):
        ones_v[pl.ds(j * 16, 16)] = jnp.ones((16,), jnp.float32)
    plsc.subcore_barrier()
    base = (c * NS + s) * EPT

    def body(i, carry):
        off = base + i * CHUNK
        pltpu.sync_copy(dst_hbm.at[pl.ds(off, CHUNK)], idx_v)
        pltpu.sync_copy(ones_v, acc_sh.at[idx_v], add=True)
        return carry

    lax.fori_loop(0, NCHUNK, body, 0)
    plsc.subcore_barrier()
    pltpu.sync_copy(acc_sh.at[pl.ds(s * RPT, RPT)],
                    out_hbm.at[c, pl.ds(s * RPT, RPT)])


@functools.partial(
    pl.kernel,
    mesh=_mesh,
    out_type=jax.ShapeDtypeStruct((NC, NPAD, H), jnp.float32),
    scratch_types=[
        pltpu.VMEM((CHUNK,), jnp.int32),
        pltpu.VMEM((CHUNK,), jnp.int32),
        pltpu.VMEM((CHUNK, H), jnp.float32),
        pltpu.VMEM_SHARED((NPAD, H), jnp.float32),
        pltpu.SemaphoreType.DMA,
    ],
)
def _sc_msg(src_hbm, dst_hbm, tab_hbm, zer_hbm, out_hbm,
            si_v, di_v, rows_v, acc_sh, sem):
    c = lax.axis_index("c")
    s = lax.axis_index("s")
    pltpu.sync_copy(zer_hbm, acc_sh.at[pl.ds(s * RPT, RPT)])
    plsc.subcore_barrier()
    base = (c * NS + s) * EPT

    def body(i, carry):
        off = base + i * CHUNK
        pltpu.sync_copy(src_hbm.at[pl.ds(off, CHUNK)], si_v)
        pltpu.sync_copy(dst_hbm.at[pl.ds(off, CHUNK)], di_v)
        pltpu.async_copy(tab_hbm.at[si_v], rows_v, sem).wait()
        pltpu.sync_copy(rows_v, acc_sh.at[di_v], add=True)
        return carry

    lax.fori_loop(0, NCHUNK, body, 0)
    plsc.subcore_barrier()
    pltpu.sync_copy(acc_sh.at[pl.ds(s * RPT, RPT)],
                    out_hbm.at[c, pl.ds(s * RPT, RPT)])


def _k1_body(x_ref, w_ref, d0_ref, d1_ref, t_ref):
    dinv = lax.rsqrt(d0_ref[...] + d1_ref[...] + 1.0)
    t_ref[...] = jnp.dot(x_ref[...], w_ref[...],
                         preferred_element_type=jnp.float32) * dinv


def _k1(x, w, d0, d1):
    return pl.pallas_call(
        _k1_body,
        grid=(N // BK,),
        in_specs=[
            pl.BlockSpec((BK, H), lambda i: (i, 0)),
            pl.BlockSpec((H, H), lambda i: (0, 0)),
            pl.BlockSpec((BK, 1), lambda i: (i, 0)),
            pl.BlockSpec((BK, 1), lambda i: (i, 0)),
        ],
        out_specs=pl.BlockSpec((BK, H), lambda i: (i, 0)),
        out_shape=jax.ShapeDtypeStruct((N, H), jnp.float32),
    )(x, w, d0, d1)


def _k2_body(a0_ref, a1_ref, t_ref, d0_ref, d1_ref, b_ref, z_ref, st_ref):
    i = pl.program_id(0)
    dinv = lax.rsqrt(d0_ref[...] + d1_ref[...] + 1.0)
    z = dinv * (a0_ref[...] + a1_ref[...] + t_ref[...]) + b_ref[...]
    z_ref[...] = z
    ssum = jnp.sum(z, axis=0, keepdims=True)
    ssq = jnp.sum(z * z, axis=0, keepdims=True)
    blk = jnp.concatenate([ssum, ssq, jnp.zeros((6, H), jnp.float32)], axis=0)

    @pl.when(i == 0)
    def _():
        st_ref[...] = blk

    @pl.when(i > 0)
    def _():
        st_ref[...] += blk


def _k2(a0, a1, t, d0, d1, b):
    return pl.pallas_call(
        _k2_body,
        grid=(N // BK,),
        in_specs=[
            pl.BlockSpec((BK, H), lambda i: (i, 0)),
            pl.BlockSpec((BK, H), lambda i: (i, 0)),
            pl.BlockSpec((BK, H), lambda i: (i, 0)),
            pl.BlockSpec((BK, 1), lambda i: (i, 0)),
            pl.BlockSpec((BK, 1), lambda i: (i, 0)),
            pl.BlockSpec((1, H), lambda i: (0, 0)),
        ],
        out_specs=[
            pl.BlockSpec((BK, H), lambda i: (i, 0)),
            pl.BlockSpec((8, H), lambda i: (0, 0)),
        ],
        out_shape=[
            jax.ShapeDtypeStruct((N, H), jnp.float32),
            jax.ShapeDtypeStruct((8, H), jnp.float32),
        ],
    )(a0, a1, t, d0, d1, b)


def _k3_body(z_ref, st_ref, g_ref, be_ref, d0_ref, d1_ref, w_ref, t_ref):
    mean = st_ref[0:1, :] * (1.0 / N)
    var = st_ref[1:2, :] * (1.0 / N) - mean * mean
    rstd = lax.rsqrt(var + 1e-5)
    h = jnp.maximum((z_ref[...] - mean) * rstd * g_ref[...] + be_ref[...], 0.0)
    dinv = lax.rsqrt(d0_ref[...] + d1_ref[...] + 1.0)
    t_ref[...] = jnp.dot(h, w_ref[...],
                         preferred_element_type=jnp.float32) * dinv


def _k3(z, st, g, be, d0, d1, w):
    return pl.pallas_call(
        _k3_body,
        grid=(N // BK,),
        in_specs=[
            pl.BlockSpec((BK, H), lambda i: (i, 0)),
            pl.BlockSpec((8, H), lambda i: (0, 0)),
            pl.BlockSpec((1, H), lambda i: (0, 0)),
            pl.BlockSpec((1, H), lambda i: (0, 0)),
            pl.BlockSpec((BK, 1), lambda i: (i, 0)),
            pl.BlockSpec((BK, 1), lambda i: (i, 0)),
            pl.BlockSpec((H, H), lambda i: (0, 0)),
        ],
        out_specs=pl.BlockSpec((BK, H), lambda i: (i, 0)),
        out_shape=jax.ShapeDtypeStruct((N, H), jnp.float32),
    )(z, st, g, be, d0, d1, w)


def _k5_body(z_ref, st_ref, g_ref, be_ref, bt_ref, s_ref, c_ref):
    i = pl.program_id(0)
    mean = st_ref[0:1, :] * (1.0 / N)
    var = st_ref[1:2, :] * (1.0 / N) - mean * mean
    rstd = lax.rsqrt(var + 1e-5)
    h = jnp.maximum((z_ref[...] - mean) * rstd * g_ref[...] + be_ref[...], 0.0)
    lanes = lax.broadcasted_iota(jnp.int32, (BK, 128), 1)
    onehot = (bt_ref[...] == lanes).astype(jnp.float32)
    sblk = lax.dot_general(onehot, h, (((0,), (0,)), ((), ())),
                           preferred_element_type=jnp.float32)
    cblk = lax.dot_general(onehot, jnp.ones((BK, 128), jnp.float32),
                           (((0,), (0,)), ((), ())),
                           preferred_element_type=jnp.float32)

    @pl.when(i == 0)
    def _():
        s_ref[...] = sblk
        c_ref[...] = cblk

    @pl.when(i > 0)
    def _():
        s_ref[...] += sblk
        c_ref[...] += cblk


def _k5(z, st, g, be, bt):
    return pl.pallas_call(
        _k5_body,
        grid=(N // BK,),
        in_specs=[
            pl.BlockSpec((BK, H), lambda i: (i, 0)),
            pl.BlockSpec((8, H), lambda i: (0, 0)),
            pl.BlockSpec((1, H), lambda i: (0, 0)),
            pl.BlockSpec((1, H), lambda i: (0, 0)),
            pl.BlockSpec((BK, 1), lambda i: (i, 0)),
        ],
        out_specs=[
            pl.BlockSpec((128, 128), lambda i: (0, 0)),
            pl.BlockSpec((128, 128), lambda i: (0, 0)),
        ],
        out_shape=[
            jax.ShapeDtypeStruct((128, 128), jnp.float32),
            jax.ShapeDtypeStruct((128, 128), jnp.float32),
        ],
    )(z, st, g, be, bt)


def _k6_body(s_ref, c_ref, w_ref, b_ref, o_ref):
    pooled = s_ref[...] / jnp.maximum(c_ref[...], 1.0)
    o_ref[...] = jnp.dot(pooled, w_ref[...],
                         preferred_element_type=jnp.float32) + b_ref[...]


def _k6(s, cnt, w, b):
    return pl.pallas_call(
        _k6_body,
        out_shape=jax.ShapeDtypeStruct((128, 128), jnp.float32),
    )(s, cnt, w, b)


def kernel(x, edge_index, batch, W1, b1, g1, be1, W2, b2, g2, be2, fcW, fcb):
    src = edge_index[0]
    dst = edge_index[1]
    zer1 = jnp.zeros((RPT,), jnp.float32)
    zer2 = jnp.zeros((RPT, H), jnp.float32)

    degp = _sc_deg(dst, zer1)
    d0 = degp[0, :N].reshape(N, 1)
    d1 = degp[1, :N].reshape(N, 1)

    t1 = _k1(x, W1, d0, d1)
    accp = _sc_msg(src, dst, t1, zer2)
    z1, st1 = _k2(accp[0, :N], accp[1, :N], t1, d0, d1, b1.reshape(1, H))

    t2 = _k3(z1, st1, g1.reshape(1, H), be1.reshape(1, H), d0, d1, W2)
    accp2 = _sc_msg(src, dst, t2, zer2)
    z2, st2 = _k2(accp2[0, :N], accp2[1, :N], t2, d0, d1, b2.reshape(1, H))

    s, cnt = _k5(z2, st2, g2.reshape(1, H), be2.reshape(1, H),
                 batch.reshape(N, 1))
    wp = jnp.pad(fcW, ((0, 0), (0, 128 - C)))
    bp = jnp.pad(fcb, (0, 128 - C)).reshape(1, 128)
    out = _k6(s, cnt, wp, bp)
    return out[:G, :C]
